# baseline (speedup 1.0000x reference)
import functools

import jax
import jax.numpy as jnp
from jax import lax
from jax.experimental import pallas as pl
from jax.experimental.pallas import tpu as pltpu
from jax.experimental.pallas import tpu_sc as plsc

LANES = 16


@functools.lru_cache(maxsize=None)
def _build_transpose(V, D, NC, NS):
    NW = NC * NS
    CW = 512
    n_full = V // CW
    rem = V - n_full * CW
    rem128 = rem // 128 * 128
    tail = rem - rem128
    per_tile = -(-n_full // NW)

    mesh = plsc.VectorSubcoreMesh(core_axis_name="c", subcore_axis_name="s")

    NG = D // 8
    out_type = jax.ShapeDtypeStruct((V * D,), jnp.float32)

    @functools.partial(
        pl.kernel,
        mesh=mesh,
        out_type=out_type,
        scratch_types=[
            pltpu.VMEM((NG, 8, CW), jnp.float32),
            pltpu.VMEM((NG, 8, CW), jnp.float32),
            pltpu.VMEM((CW * D,), jnp.float32),
            pltpu.VMEM((CW * D,), jnp.float32),
            pltpu.VMEM((D // 8, 8, 128), jnp.float32),
            pltpu.SemaphoreType.DMA,
            pltpu.SemaphoreType.DMA,
            pltpu.SemaphoreType.DMA,
            pltpu.SemaphoreType.DMA,
        ],
        compiler_params=pltpu.CompilerParams(needs_layout_passes=False),
    )
    def trans(wt_hbm, tail_hbm, wv_hbm, in0, in1, out0, out1, in_r,
              si0, si1, so0, so1):
        wid = lax.axis_index("s") * NC + lax.axis_index("c")
        iota = lax.iota(jnp.int32, LANES)
        sc_idx = iota * D
        ins = (in0, in1)
        outs = (out0, out1)
        sis = (si0, si1)
        sos = (so0, so1)

        def col_of(k):
            return k * NW + wid

        def fire_in(k):
            p = k % 2

            @pl.when(col_of(k) < n_full)
            def _():
                v0 = pl.multiple_of(col_of(k) * CW, 128)
                for g in range(NG):
                    pltpu.async_copy(
                        wt_hbm.at[g, pl.ds(0, 8), pl.ds(v0, CW)],
                        ins[p].at[g],
                        sis[p],
                    )

        def drain_in(k):
            p = k % 2

            @pl.when(col_of(k) < n_full)
            def _():
                for _g in range(NG):
                    pltpu.make_async_copy(
                        wt_hbm.at[0, pl.ds(0, 8), pl.ds(0, CW)], ins[p].at[0], sis[p]
                    ).wait()

        fire_in(0)
        for k in range(per_tile):
            c = col_of(k)
            if k + 1 < per_tile:
                fire_in(k + 1)
            drain_in(k)

            @pl.when(c < n_full)
            def _(k=k, c=c):
                p = k % 2
                iv, ov = ins[p], outs[p]
                if k >= 2:
                    pltpu.make_async_copy(
                        ov, wv_hbm.at[pl.ds(0, CW * D)], sos[p]
                    ).wait()

                @plsc.parallel_loop(0, CW // LANES)
                def colblk(jb):
                    j0 = jb * LANES
                    for d in range(D):
                        vals = iv[d // 8, d % 8, pl.ds(j0, LANES)]
                        plsc.store_scatter(ov, [sc_idx + (j0 * D + d)], vals)

                v0 = pl.multiple_of(c * CW, 128)
                pltpu.async_copy(ov, wv_hbm.at[pl.ds(v0 * D, CW * D)], sos[p])

        for k in range(max(0, per_tile - 2), per_tile):
            @pl.when(col_of(k) < n_full)
            def _(k=k):
                p = k % 2
                pltpu.make_async_copy(
                    outs[p], wv_hbm.at[pl.ds(0, CW * D)], sos[p]
                ).wait()

        if rem128:
            @pl.when(wid == NW - 2)
            def _():
                v0 = n_full * CW
                for g in range(NG):
                    pltpu.sync_copy(
                        wt_hbm.at[g, pl.ds(0, 8), pl.ds(v0, rem128)],
                        in_r.at[g],
                    )

                @plsc.parallel_loop(0, rem128 // LANES)
                def colblk(jb):
                    j0 = jb * LANES
                    for d in range(D):
                        vals = in_r[d // 8, d % 8, pl.ds(j0, LANES)]
                        plsc.store_scatter(out0, [sc_idx + (j0 * D + d)], vals)

                pltpu.sync_copy(
                    out0.at[pl.ds(0, rem128 * D)],
                    wv_hbm.at[pl.ds(v0 * D, rem128 * D)],
                )

        if tail:
            @pl.when(wid == NW - 1)
            def _():
                pltpu.sync_copy(tail_hbm, out1.at[pl.ds(0, tail * D)])
                pltpu.sync_copy(
                    out1.at[pl.ds(0, tail * D)],
                    wv_hbm.at[pl.ds((n_full * CW + rem128) * D, tail * D)],
                )

    return trans


@functools.lru_cache(maxsize=None)
def _build_gather(B, H, V, D, NC, NS):
    NW = NC * NS
    b_per_w = B // NW
    CH = 64
    n_chunks = b_per_w // CH
    CR = CH * H

    mesh = plsc.VectorSubcoreMesh(core_axis_name="c", subcore_axis_name="s")

    @functools.partial(
        pl.kernel,
        mesh=mesh,
        out_type=jax.ShapeDtypeStruct((B, D), jnp.float32),
        scratch_types=[
            pltpu.VMEM((H, b_per_w), jnp.int32),
            pltpu.VMEM((CR, D), jnp.float32),
            pltpu.VMEM((CR, D), jnp.float32),
            pltpu.VMEM((CH, D), jnp.float32),
            pltpu.SemaphoreType.DMA,
            pltpu.SemaphoreType.DMA,
        ],
        compiler_params=pltpu.CompilerParams(
            use_tc_tiling_on_sc=False, needs_layout_passes=False
        ),
    )
    def emb(idx_hbm, wv_hbm, out_hbm, idx_v, rows0, rows1, out_v, sem0, sem1):
        wid = lax.axis_index("s") * NC + lax.axis_index("c")
        base_b = wid * b_per_w
        stage = [
            pltpu.async_copy(
                idx_hbm.at[pl.ds(pl.multiple_of(h * B, 8) + base_b, b_per_w)],
                idx_v.at[h],
                sem0,
            )
            for h in range(H)
        ]
        for cp in stage:
            cp.wait()

        rows = (rows0, rows1)
        sems = (sem0, sem1)

        def fire(c):
            k = c % 2
            return [
                pltpu.async_copy(
                    wv_hbm.at[idx_v.at[h, pl.ds(c * CH, CH)]],
                    rows[k].at[pl.ds(h * CH, CH)],
                    sems[k],
                )
                for h in range(H)
            ]

        pending = fire(0)
        for c in range(n_chunks):
            nxt = fire(c + 1) if c + 1 < n_chunks else None
            for cp in pending:
                cp.wait()
            rv = rows[c % 2]

            @plsc.parallel_loop(0, CH)
            def reduce_row(i):
                a0 = rv[i, pl.ds(0, LANES)]
                a1 = rv[i, pl.ds(LANES, LANES)]
                for h in range(1, H):
                    a0 = a0 + rv[h * CH + i, pl.ds(0, LANES)]
                    a1 = a1 + rv[h * CH + i, pl.ds(LANES, LANES)]
                out_v[i, pl.ds(0, LANES)] = a0
                out_v[i, pl.ds(LANES, LANES)] = a1

            pltpu.sync_copy(out_v, out_hbm.at[pl.ds(base_b + c * CH, CH)])
            pending = nxt

    return emb


def kernel(input, weight):
    B, H = input.shape
    V, D = weight.shape
    info = plsc.get_sparse_core_info()
    trans = _build_transpose(V, D, info.num_cores, info.num_subcores)
    emb = _build_gather(B, H, V, D, info.num_cores, info.num_subcores)
    tail = V % 128
    tail_lin = weight[V - tail:, :].reshape(tail * D)
    wv = trans(weight.T.reshape(D // 8, 8, V), tail_lin)
    return emb(input.T.reshape(H * B), wv.reshape(V, D))


# speedup vs baseline: 1.3652x; 1.3652x over previous
import functools

import jax
import jax.numpy as jnp
from jax import lax
from jax.experimental import pallas as pl
from jax.experimental.pallas import tpu as pltpu
from jax.experimental.pallas import tpu_sc as plsc

LANES = 16


@functools.lru_cache(maxsize=None)
def _build_transpose(V, D, NC, NS):
    NW = NC * NS
    CW = 512
    n_full = V // CW
    rem = V - n_full * CW
    rem128 = rem // 128 * 128
    tail = rem - rem128
    per_tile = -(-n_full // NW)

    mesh = plsc.VectorSubcoreMesh(core_axis_name="c", subcore_axis_name="s")

    NG = D // 8
    out_type = jax.ShapeDtypeStruct((V * D,), jnp.float32)

    @functools.partial(
        pl.kernel,
        mesh=mesh,
        out_type=out_type,
        scratch_types=[
            pltpu.VMEM((D, CW), jnp.float32),
            pltpu.VMEM((D, CW), jnp.float32),
            pltpu.VMEM((CW * D,), jnp.float32),
            pltpu.VMEM((CW * D,), jnp.float32),
            pltpu.VMEM((D, 128), jnp.float32),
            pltpu.VMEM((128 * D,), jnp.float32),
            pltpu.SemaphoreType.DMA,
            pltpu.SemaphoreType.DMA,
            pltpu.SemaphoreType.DMA,
            pltpu.SemaphoreType.DMA,
        ],
        compiler_params=pltpu.CompilerParams(needs_layout_passes=False),
    )
    def trans(wt_hbm, tail_hbm, wv_hbm, in0, in1, out0, out1, in_r, out_r,
              si0, si1, so0, so1):
        wid = lax.axis_index("s") * NC + lax.axis_index("c")
        iota = lax.iota(jnp.int32, LANES)
        mrolls = [(iota + r) % LANES for r in range(LANES)]
        ins = (in0, in1)
        outs = (out0, out1)
        sis = (si0, si1)
        sos = (so0, so1)

        def col_of(k):
            return k * NW + wid

        def fire_in(k):
            p = k % 2

            @pl.when(col_of(k) < n_full)
            def _():
                v0 = pl.multiple_of(col_of(k) * CW, 128)
                for g in range(NG):
                    pltpu.async_copy(
                        wt_hbm.at[g, pl.ds(0, 8), pl.ds(v0, CW)],
                        ins[p].at[pl.ds(g * 8, 8)],
                        sis[p],
                    )

        def drain_in(k):
            p = k % 2

            @pl.when(col_of(k) < n_full)
            def _():
                for _g in range(NG):
                    pltpu.make_async_copy(
                        wt_hbm.at[0, pl.ds(0, 8), pl.ds(0, CW)],
                        ins[p].at[pl.ds(0, 8)],
                        sis[p],
                    ).wait()

        fire_in(0)
        for k in range(per_tile):
            c = col_of(k)
            if k + 1 < per_tile:
                fire_in(k + 1)
            drain_in(k)

            @pl.when(c < n_full)
            def _(k=k, c=c):
                p = k % 2
                iv, ov = ins[p], outs[p]
                if k >= 2:
                    pltpu.make_async_copy(
                        ov, wv_hbm.at[pl.ds(0, CW * D)], sos[p]
                    ).wait()

                @plsc.parallel_loop(0, CW // LANES)
                def colblk(jb):
                    j0 = jb * LANES
                    for d0 in range(0, D, LANES):
                        rows = iota + d0
                        for r in range(LANES):
                            m = mrolls[r]
                            vals = plsc.load_gather(iv, [rows, m + j0])
                            plsc.store_scatter(
                                ov, [m * D + (iota + (j0 * D + d0))], vals
                            )

                v0 = pl.multiple_of(c * CW, 128)
                pltpu.async_copy(ov, wv_hbm.at[pl.ds(v0 * D, CW * D)], sos[p])

        for k in range(max(0, per_tile - 2), per_tile):
            @pl.when(col_of(k) < n_full)
            def _(k=k):
                p = k % 2
                pltpu.make_async_copy(
                    outs[p], wv_hbm.at[pl.ds(0, CW * D)], sos[p]
                ).wait()

        if rem128:
            @pl.when(wid == NW - 2)
            def _():
                v0 = n_full * CW
                for g in range(NG):
                    pltpu.sync_copy(
                        wt_hbm.at[g, pl.ds(0, 8), pl.ds(v0, rem128)],
                        in_r.at[pl.ds(g * 8, 8)],
                    )

                @plsc.parallel_loop(0, rem128 // LANES)
                def colblk(jb):
                    j0 = jb * LANES
                    for d0 in range(0, D, LANES):
                        rows = iota + d0
                        for r in range(LANES):
                            m = mrolls[r]
                            vals = plsc.load_gather(in_r, [rows, m + j0])
                            plsc.store_scatter(
                                out_r, [m * D + (iota + (j0 * D + d0))], vals
                            )

                pltpu.sync_copy(out_r, wv_hbm.at[pl.ds(v0 * D, rem128 * D)])

        if tail:
            @pl.when(wid == NW - 1)
            def _():
                pltpu.sync_copy(tail_hbm, out_r.at[pl.ds(0, tail * D)])
                pltpu.sync_copy(
                    out_r.at[pl.ds(0, tail * D)],
                    wv_hbm.at[pl.ds((n_full * CW + rem128) * D, tail * D)],
                )

    return trans


@functools.lru_cache(maxsize=None)
def _build_gather(B, H, V, D, NC, NS):
    NW = NC * NS
    b_per_w = B // NW
    CH = 64
    n_chunks = b_per_w // CH
    CR = CH * H

    mesh = plsc.VectorSubcoreMesh(core_axis_name="c", subcore_axis_name="s")

    @functools.partial(
        pl.kernel,
        mesh=mesh,
        out_type=jax.ShapeDtypeStruct((B, D), jnp.float32),
        scratch_types=[
            pltpu.VMEM((H, b_per_w), jnp.int32),
            pltpu.VMEM((CR, D), jnp.float32),
            pltpu.VMEM((CR, D), jnp.float32),
            pltpu.VMEM((CH, D), jnp.float32),
            pltpu.SemaphoreType.DMA,
            pltpu.SemaphoreType.DMA,
        ],
        compiler_params=pltpu.CompilerParams(
            use_tc_tiling_on_sc=False, needs_layout_passes=False
        ),
    )
    def emb(idx_hbm, wv_hbm, out_hbm, idx_v, rows0, rows1, out_v, sem0, sem1):
        wid = lax.axis_index("s") * NC + lax.axis_index("c")
        base_b = wid * b_per_w
        stage = [
            pltpu.async_copy(
                idx_hbm.at[pl.ds(pl.multiple_of(h * B, 8) + base_b, b_per_w)],
                idx_v.at[h],
                sem0,
            )
            for h in range(H)
        ]
        for cp in stage:
            cp.wait()

        rows = (rows0, rows1)
        sems = (sem0, sem1)

        def fire(c):
            k = c % 2
            return [
                pltpu.async_copy(
                    wv_hbm.at[idx_v.at[h, pl.ds(c * CH, CH)]],
                    rows[k].at[pl.ds(h * CH, CH)],
                    sems[k],
                )
                for h in range(H)
            ]

        pending = fire(0)
        for c in range(n_chunks):
            nxt = fire(c + 1) if c + 1 < n_chunks else None
            for cp in pending:
                cp.wait()
            rv = rows[c % 2]

            @plsc.parallel_loop(0, CH)
            def reduce_row(i):
                a0 = rv[i, pl.ds(0, LANES)]
                a1 = rv[i, pl.ds(LANES, LANES)]
                for h in range(1, H):
                    a0 = a0 + rv[h * CH + i, pl.ds(0, LANES)]
                    a1 = a1 + rv[h * CH + i, pl.ds(LANES, LANES)]
                out_v[i, pl.ds(0, LANES)] = a0
                out_v[i, pl.ds(LANES, LANES)] = a1

            pltpu.sync_copy(out_v, out_hbm.at[pl.ds(base_b + c * CH, CH)])
            pending = nxt

    return emb


def kernel(input, weight):
    B, H = input.shape
    V, D = weight.shape
    info = plsc.get_sparse_core_info()
    trans = _build_transpose(V, D, info.num_cores, info.num_subcores)
    emb = _build_gather(B, H, V, D, info.num_cores, info.num_subcores)
    tail = V % 128
    tail_lin = weight[V - tail:, :].reshape(tail * D)
    wv = trans(weight.T.reshape(D // 8, 8, V), tail_lin)
    return emb(input.T.reshape(H * B), wv.reshape(V, D))
